# baseline (speedup 1.0000x reference)
import functools

import jax
import jax.numpy as jnp
from jax import lax
from jax.experimental import pallas as pl
from jax.experimental.pallas import tpu as pltpu
from jax.experimental.pallas import tpu_sc as plsc

N = 4096
KNN = 20
_PREC = jax.lax.Precision.DEFAULT

_SC_CORES = 2
_SC_SUBCORES = 16
_SC_WORKERS = _SC_CORES * _SC_SUBCORES


def _dot(a, b):
    return jnp.dot(a, b, precision=_PREC, preferred_element_type=jnp.float32)


WSEG = 1408


def _knn(x_like_ref, xt_scr, sqc_scr, d_scr, brow_ref, bcol_ref, seg_ref,
         idx_ref, R):
    xt = xt_scr[:]
    sqc_scr[:] = jnp.sum(xt * xt, axis=0, keepdims=True)
    col_iota = lax.broadcasted_iota(jnp.int32, (R, WSEG), 1)
    k_iota = lax.broadcasted_iota(jnp.int32, (R, KNN), 1)

    for b in range(4):
        w0 = pl.multiple_of(seg_ref[b], 128)
        xtw = xt_scr[:, pl.ds(w0, WSEG)]
        sq_col = sqc_scr[:, pl.ds(w0, WSEG)]
        bcol = bcol_ref[:, pl.ds(w0, WSEG)]

        def block(s, _):
            r0 = pl.multiple_of(jnp.minimum(w0 + s * R, N - R), 128)
            xr = x_like_ref[pl.ds(r0, R), :]
            mm = _dot(xr, xtw)
            sq_row = jnp.sum(xr * xr, axis=1, keepdims=True)
            d = (sq_row + sq_col) - 2.0 * mm
            brow = brow_ref[pl.ds(r0, R), :]
            d_scr[:] = jnp.where(brow != bcol, jnp.float32(1e10), d)

            def pick(t, carry):
                acc, sel_prev = carry
                dd = jnp.where(col_iota == sel_prev, jnp.float32(jnp.inf),
                               d_scr[:])
                d_scr[:] = dd
                m = jnp.min(dd, axis=1, keepdims=True)
                sel = jnp.min(jnp.where(dd == m, col_iota, jnp.int32(2**30)),
                              axis=1, keepdims=True)
                acc = jnp.where(k_iota == t, sel, acc)
                return acc, sel

            acc, _ = lax.fori_loop(
                0, KNN, pick,
                (jnp.zeros((R, KNN), jnp.int32),
                 jnp.full((R, 1), -1, jnp.int32)))
            old = idx_ref[pl.ds(r0, R), :]
            keep = brow == b
            idx_ref[pl.ds(r0, R), :] = jnp.where(keep, acc + w0, old)
            return 0

        lax.fori_loop(0, seg_ref[4 + b], block, 0)


def _edge_max(x, a, xj_hbm, wb, d_in, slab, sems):
    def cp(j, buf):
        return pltpu.make_async_copy(xj_hbm.at[pl.ds(j * N, N)],
                                     slab.at[buf], sems.at[buf])

    cp(0, 0).start()

    def step(j, acc):
        buf = j % 2

        @pl.when(j + 1 < KNN)
        def _():
            cp(j + 1, 1 - buf).start()

        cp(j, buf).wait()
        xj = slab[buf][:, 0:d_in]
        msg = jnp.maximum(a + _dot(xj - x, wb), 0.0)
        return jnp.maximum(acc, msg)

    init = jnp.full((N, a.shape[1]), -jnp.inf, jnp.float32)
    return lax.fori_loop(0, KNN, step, init)


def _first_body(pos_ref, brow_ref, bcol_ref, w0_ref, w1_ref, b1_ref, wl1_ref,
                bl1_ref, idx_ref, x_ref, a_ref, x64_scr, xt_scr, sqc_scr,
                d_scr):
    x = jnp.maximum(_dot(pos_ref[:], w1_ref[:]) + b1_ref[:], 0.0)
    x_ref[:, 0:64] = x
    x_ref[:, 64:128] = jnp.zeros((N, 64), jnp.float32)
    x64_scr[:] = x
    xt_scr[:] = x.T
    a_ref[:] = _dot(x, wl1_ref[0:64, :]) + bl1_ref[:]
    _knn(x64_scr, xt_scr, sqc_scr, d_scr, brow_ref, bcol_ref, w0_ref,
         idx_ref, 1024)


def _mid_body(x1_ref, a1_ref, xj_ref, brow_ref, bcol_ref, w0_ref, wl1_ref,
              wl2_ref, bl2_ref, idx_ref, x_ref, a_ref, xt_scr, sqc_scr,
              d_scr, slab, sems):
    x2 = _edge_max(x1_ref[:, 0:64], a1_ref[:], xj_ref, wl1_ref[64:128, :],
                   64, slab, sems)
    x_ref[:] = x2
    xt_scr[:] = x2.T
    a_ref[:] = _dot(x2, wl2_ref[0:128, :]) + bl2_ref[:]
    _knn(x_ref, xt_scr, sqc_scr, d_scr, brow_ref, bcol_ref, w0_ref,
         idx_ref, 1024)


def _last_body(x2_ref, a2_ref, xj_ref, wl2_ref, w2_ref, b2_ref, o_ref,
               slab, sems):
    x3 = _edge_max(x2_ref[:], a2_ref[:], xj_ref, wl2_ref[128:256, :],
                   128, slab, sems)
    o_ref[:] = _dot(x3, w2_ref[:]) + b2_ref[:]


_VMEM_SPEC = pl.BlockSpec(memory_space=pltpu.MemorySpace.VMEM)
_SMEM_SPEC = pl.BlockSpec(memory_space=pltpu.MemorySpace.SMEM)
_ANY_SPEC = pl.BlockSpec(memory_space=pl.ANY)

_TC_FIRST = pl.pallas_call(
    _first_body,
    in_specs=[_VMEM_SPEC, _VMEM_SPEC, _VMEM_SPEC, _SMEM_SPEC,
              _VMEM_SPEC, _VMEM_SPEC, _VMEM_SPEC, _VMEM_SPEC],
    out_shape=(
        jax.ShapeDtypeStruct((N, KNN), jnp.int32),
        jax.ShapeDtypeStruct((N, 128), jnp.float32),
        jax.ShapeDtypeStruct((N, 128), jnp.float32),
    ),
    scratch_shapes=[
        pltpu.VMEM((N, 64), jnp.float32),
        pltpu.VMEM((64, N), jnp.float32),
        pltpu.VMEM((1, N), jnp.float32),
        pltpu.VMEM((1024, WSEG), jnp.float32),
    ],
)

_TC_MID = pl.pallas_call(
    _mid_body,
    in_specs=[_VMEM_SPEC, _VMEM_SPEC, _ANY_SPEC, _VMEM_SPEC, _VMEM_SPEC,
              _SMEM_SPEC, _VMEM_SPEC, _VMEM_SPEC, _VMEM_SPEC],
    out_shape=(
        jax.ShapeDtypeStruct((N, KNN), jnp.int32),
        jax.ShapeDtypeStruct((N, 128), jnp.float32),
        jax.ShapeDtypeStruct((N, 256), jnp.float32),
    ),
    scratch_shapes=[
        pltpu.VMEM((128, N), jnp.float32),
        pltpu.VMEM((1, N), jnp.float32),
        pltpu.VMEM((1024, WSEG), jnp.float32),
        pltpu.VMEM((2, N, 128), jnp.float32),
        pltpu.SemaphoreType.DMA((2,)),
    ],
)

_TC_LAST = pl.pallas_call(
    _last_body,
    in_specs=[_VMEM_SPEC, _VMEM_SPEC, _ANY_SPEC,
              _VMEM_SPEC, _VMEM_SPEC, _VMEM_SPEC],
    out_shape=jax.ShapeDtypeStruct((N, 128), jnp.float32),
    scratch_shapes=[
        pltpu.VMEM((2, N, 128), jnp.float32),
        pltpu.SemaphoreType.DMA((2,)),
    ],
)


def _make_gather(D):
    G = N * KNN
    GPW = G // _SC_WORKERS
    CH = 128
    NCHUNK = GPW // CH
    mesh = plsc.VectorSubcoreMesh(core_axis_name="c", subcore_axis_name="s",
                                  num_cores=_SC_CORES,
                                  num_subcores=_SC_SUBCORES)

    @functools.partial(
        pl.kernel, mesh=mesh,
        out_type=jax.ShapeDtypeStruct((G, D), jnp.float32),
        scratch_types=[
            pltpu.VMEM((CH,), jnp.int32),
            pltpu.VMEM((CH, D), jnp.float32),
            pltpu.SemaphoreType.DMA,
        ],
    )
    def gather(idx_hbm, table_hbm, out_hbm, idx_v, rows_v, sem):
        wid = lax.axis_index("s") * _SC_CORES + lax.axis_index("c")
        base = wid * GPW

        def chunk(ci, _):
            g0 = base + ci * CH
            pltpu.sync_copy(idx_hbm.at[pl.ds(g0, CH)], idx_v)
            pltpu.async_copy(table_hbm.at[idx_v], rows_v, sem).wait()
            pltpu.sync_copy(rows_v, out_hbm.at[pl.ds(g0, CH)])
            return 0

        lax.fori_loop(0, NCHUNK, chunk, 0)

    return gather


_GATHER128 = functools.cache(lambda: _make_gather(128))


def kernel(pos, batch, W1, b1, Wl1, bl1, Wl2, bl2, W2, b2):
    brow = batch.reshape(N, 1)
    bcol = batch.reshape(1, N)
    seg_start = jnp.searchsorted(batch, jnp.arange(4, dtype=batch.dtype))
    seg_end = jnp.concatenate([seg_start[1:],
                               jnp.array([N], seg_start.dtype)])
    w0 = jnp.minimum((seg_start // 128) * 128, N - WSEG).astype(jnp.int32)
    nb = ((seg_end.astype(jnp.int32) - w0) + 1023) // 1024
    w0 = jnp.concatenate([w0, nb]).astype(jnp.int32)
    idx1, x1, A1 = _TC_FIRST(pos, brow, bcol, w0, W1, b1.reshape(1, -1),
                             Wl1, bl1.reshape(1, -1))
    XJ1 = _GATHER128()(idx1.T.reshape(-1), x1)
    idx2, x2, A2 = _TC_MID(x1, A1, XJ1, brow, bcol, w0, Wl1, Wl2,
                           bl2.reshape(1, -1))
    XJ2 = _GATHER128()(idx2.T.reshape(-1), x2)
    return _TC_LAST(x2, A2, XJ2, Wl2, W2, b2.reshape(1, -1))


# speedup vs baseline: 1.0329x; 1.0329x over previous
import functools

import jax
import jax.numpy as jnp
from jax import lax
from jax.experimental import pallas as pl
from jax.experimental.pallas import tpu as pltpu
from jax.experimental.pallas import tpu_sc as plsc

N = 4096
KNN = 20
_PREC = jax.lax.Precision.DEFAULT

_SC_CORES = 2
_SC_SUBCORES = 16
_SC_WORKERS = _SC_CORES * _SC_SUBCORES


def _dot(a, b):
    return jnp.dot(a, b, precision=_PREC, preferred_element_type=jnp.float32)


WSEG = 1408


def _knn(x_like_ref, xt_scr, sqc_scr, d_scr, brow_ref, bcol_ref, seg_ref,
         idx_ref, R):
    xt = xt_scr[:]
    sqc_scr[:] = jnp.sum(xt * xt, axis=0, keepdims=True)
    col_iota = lax.broadcasted_iota(jnp.int32, (R, WSEG), 1)
    k_iota = lax.broadcasted_iota(jnp.int32, (R, KNN), 1)

    for b in range(4):
        w0 = pl.multiple_of(seg_ref[b], 128)
        xtw = xt_scr[:, pl.ds(w0, WSEG)]
        sq_col = sqc_scr[:, pl.ds(w0, WSEG)]
        bcol = bcol_ref[:, pl.ds(w0, WSEG)]

        def block(s, _):
            r0 = pl.multiple_of(jnp.minimum(w0 + s * R, N - R), 128)
            xr = x_like_ref[pl.ds(r0, R), :]
            mm = _dot(xr, xtw)
            sq_row = jnp.sum(xr * xr, axis=1, keepdims=True)
            d = (sq_row + sq_col) - 2.0 * mm
            brow = brow_ref[pl.ds(r0, R), :]
            d_scr[:] = jnp.where(brow != bcol, jnp.float32(1e10), d)

            def pick(t, carry):
                acc, sel_prev = carry
                dd = jnp.where(col_iota == sel_prev, jnp.float32(jnp.inf),
                               d_scr[:])
                d_scr[:] = dd
                m = jnp.min(dd, axis=1, keepdims=True)
                sel = jnp.min(jnp.where(dd == m, col_iota, jnp.int32(2**30)),
                              axis=1, keepdims=True)
                acc = jnp.where(k_iota == t, sel, acc)
                return acc, sel

            acc, _ = lax.fori_loop(
                0, KNN, pick,
                (jnp.zeros((R, KNN), jnp.int32),
                 jnp.full((R, 1), -1, jnp.int32)))
            old = idx_ref[pl.ds(r0, R), :]
            keep = brow == b
            idx_ref[pl.ds(r0, R), :] = jnp.where(keep, acc + w0, old)
            return 0

        lax.fori_loop(0, seg_ref[4 + b], block, 0)


def _edge_max(x, a, xj_hbm, wb, d_in, slab, sems):
    def cp(j, buf):
        return pltpu.make_async_copy(xj_hbm.at[pl.ds(j * N, N)],
                                     slab.at[buf], sems.at[buf])

    cp(0, 0).start()

    def step(j, acc):
        buf = j % 2

        @pl.when(j + 1 < KNN)
        def _():
            cp(j + 1, 1 - buf).start()

        cp(j, buf).wait()
        xj = slab[buf][:, 0:d_in]
        msg = jnp.maximum(a + _dot(xj - x, wb), 0.0)
        return jnp.maximum(acc, msg)

    init = jnp.full((N, a.shape[1]), -jnp.inf, jnp.float32)
    return lax.fori_loop(0, KNN, step, init)


def _first_body(pos_ref, brow_ref, bcol_ref, w0_ref, w1_ref, b1_ref, wl1_ref,
                bl1_ref, idx_ref, x_ref, a_ref, x64_scr, xt_scr, sqc_scr,
                d_scr):
    x = jnp.maximum(_dot(pos_ref[:], w1_ref[:]) + b1_ref[:], 0.0)
    x_ref[:, 0:64] = x
    x_ref[:, 64:128] = jnp.zeros((N, 64), jnp.float32)
    x64_scr[:] = x
    xt_scr[:] = x.T
    a_ref[:] = _dot(x, wl1_ref[0:64, :]) + bl1_ref[:]
    _knn(x64_scr, xt_scr, sqc_scr, d_scr, brow_ref, bcol_ref, w0_ref,
         idx_ref, 640)


def _mid_body(x1_ref, a1_ref, xj_ref, brow_ref, bcol_ref, w0_ref, wl1_ref,
              wl2_ref, bl2_ref, idx_ref, x_ref, a_ref, xt_scr, sqc_scr,
              d_scr, slab, sems):
    x2 = _edge_max(x1_ref[:, 0:64], a1_ref[:], xj_ref, wl1_ref[64:128, :],
                   64, slab, sems)
    x_ref[:] = x2
    xt_scr[:] = x2.T
    a_ref[:] = _dot(x2, wl2_ref[0:128, :]) + bl2_ref[:]
    _knn(x_ref, xt_scr, sqc_scr, d_scr, brow_ref, bcol_ref, w0_ref,
         idx_ref, 640)


def _last_body(x2_ref, a2_ref, xj_ref, wl2_ref, w2_ref, b2_ref, o_ref,
               slab, sems):
    x3 = _edge_max(x2_ref[:], a2_ref[:], xj_ref, wl2_ref[128:256, :],
                   128, slab, sems)
    o_ref[:] = _dot(x3, w2_ref[:]) + b2_ref[:]


_VMEM_SPEC = pl.BlockSpec(memory_space=pltpu.MemorySpace.VMEM)
_SMEM_SPEC = pl.BlockSpec(memory_space=pltpu.MemorySpace.SMEM)
_ANY_SPEC = pl.BlockSpec(memory_space=pl.ANY)

_TC_FIRST = pl.pallas_call(
    _first_body,
    in_specs=[_VMEM_SPEC, _VMEM_SPEC, _VMEM_SPEC, _SMEM_SPEC,
              _VMEM_SPEC, _VMEM_SPEC, _VMEM_SPEC, _VMEM_SPEC],
    out_shape=(
        jax.ShapeDtypeStruct((N, KNN), jnp.int32),
        jax.ShapeDtypeStruct((N, 128), jnp.float32),
        jax.ShapeDtypeStruct((N, 128), jnp.float32),
    ),
    scratch_shapes=[
        pltpu.VMEM((N, 64), jnp.float32),
        pltpu.VMEM((64, N), jnp.float32),
        pltpu.VMEM((1, N), jnp.float32),
        pltpu.VMEM((640, WSEG), jnp.float32),
    ],
)

_TC_MID = pl.pallas_call(
    _mid_body,
    in_specs=[_VMEM_SPEC, _VMEM_SPEC, _ANY_SPEC, _VMEM_SPEC, _VMEM_SPEC,
              _SMEM_SPEC, _VMEM_SPEC, _VMEM_SPEC, _VMEM_SPEC],
    out_shape=(
        jax.ShapeDtypeStruct((N, KNN), jnp.int32),
        jax.ShapeDtypeStruct((N, 128), jnp.float32),
        jax.ShapeDtypeStruct((N, 256), jnp.float32),
    ),
    scratch_shapes=[
        pltpu.VMEM((128, N), jnp.float32),
        pltpu.VMEM((1, N), jnp.float32),
        pltpu.VMEM((640, WSEG), jnp.float32),
        pltpu.VMEM((2, N, 128), jnp.float32),
        pltpu.SemaphoreType.DMA((2,)),
    ],
)

_TC_LAST = pl.pallas_call(
    _last_body,
    in_specs=[_VMEM_SPEC, _VMEM_SPEC, _ANY_SPEC,
              _VMEM_SPEC, _VMEM_SPEC, _VMEM_SPEC],
    out_shape=jax.ShapeDtypeStruct((N, 128), jnp.float32),
    scratch_shapes=[
        pltpu.VMEM((2, N, 128), jnp.float32),
        pltpu.SemaphoreType.DMA((2,)),
    ],
)


def _make_gather(D):
    G = N * KNN
    GPW = G // _SC_WORKERS
    CH = 128
    NCHUNK = GPW // CH
    mesh = plsc.VectorSubcoreMesh(core_axis_name="c", subcore_axis_name="s",
                                  num_cores=_SC_CORES,
                                  num_subcores=_SC_SUBCORES)

    @functools.partial(
        pl.kernel, mesh=mesh,
        out_type=jax.ShapeDtypeStruct((G, D), jnp.float32),
        scratch_types=[
            pltpu.VMEM((CH,), jnp.int32),
            pltpu.VMEM((CH, D), jnp.float32),
            pltpu.SemaphoreType.DMA,
        ],
    )
    def gather(idx_hbm, table_hbm, out_hbm, idx_v, rows_v, sem):
        wid = lax.axis_index("s") * _SC_CORES + lax.axis_index("c")
        base = wid * GPW

        def chunk(ci, _):
            g0 = base + ci * CH
            pltpu.sync_copy(idx_hbm.at[pl.ds(g0, CH)], idx_v)
            pltpu.async_copy(table_hbm.at[idx_v], rows_v, sem).wait()
            pltpu.sync_copy(rows_v, out_hbm.at[pl.ds(g0, CH)])
            return 0

        lax.fori_loop(0, NCHUNK, chunk, 0)

    return gather


_GATHER128 = functools.cache(lambda: _make_gather(128))


def kernel(pos, batch, W1, b1, Wl1, bl1, Wl2, bl2, W2, b2):
    brow = batch.reshape(N, 1)
    bcol = batch.reshape(1, N)
    seg_start = jnp.searchsorted(batch, jnp.arange(4, dtype=batch.dtype))
    seg_end = jnp.concatenate([seg_start[1:],
                               jnp.array([N], seg_start.dtype)])
    w0 = jnp.minimum((seg_start // 128) * 128, N - WSEG).astype(jnp.int32)
    nb = ((seg_end.astype(jnp.int32) - w0) + 639) // 640
    w0 = jnp.concatenate([w0, nb]).astype(jnp.int32)
    idx1, x1, A1 = _TC_FIRST(pos, brow, bcol, w0, W1, b1.reshape(1, -1),
                             Wl1, bl1.reshape(1, -1))
    XJ1 = _GATHER128()(idx1.T.reshape(-1), x1)
    idx2, x2, A2 = _TC_MID(x1, A1, XJ1, brow, bcol, w0, Wl1, Wl2,
                           bl2.reshape(1, -1))
    XJ2 = _GATHER128()(idx2.T.reshape(-1), x2)
    return _TC_LAST(x2, A2, XJ2, Wl2, W2, b2.reshape(1, -1))


# speedup vs baseline: 1.1174x; 1.0818x over previous
import functools

import jax
import jax.numpy as jnp
from jax import lax
from jax.experimental import pallas as pl
from jax.experimental.pallas import tpu as pltpu
from jax.experimental.pallas import tpu_sc as plsc

N = 4096
KNN = 20
_PREC = jax.lax.Precision.DEFAULT

_SC_CORES = 2
_SC_SUBCORES = 16
_SC_WORKERS = _SC_CORES * _SC_SUBCORES


def _dot(a, b):
    return jnp.dot(a, b, precision=_PREC, preferred_element_type=jnp.float32)


WSEG = 1408


def _knn(x_like_ref, xt_scr, sqc_scr, d_scr, brow_ref, bcol_ref, seg_ref,
         idx_ref, R):
    xt = xt_scr[:]
    sqc_scr[:] = jnp.sum(xt * xt, axis=0, keepdims=True)
    col_iota = lax.broadcasted_iota(jnp.int32, (R, WSEG), 1)
    k_iota = lax.broadcasted_iota(jnp.int32, (R, KNN), 1)

    for b in range(4):
        w0 = pl.multiple_of(seg_ref[b], 128)
        xtw = xt_scr[:, pl.ds(w0, WSEG)]
        sq_col = sqc_scr[:, pl.ds(w0, WSEG)]
        bcol = bcol_ref[:, pl.ds(w0, WSEG)]

        def block(s, _):
            r0 = pl.multiple_of(jnp.minimum(w0 + s * R, N - R), 128)
            xr = x_like_ref[pl.ds(r0, R), :]
            mm = _dot(xr, xtw)
            sq_row = jnp.sum(xr * xr, axis=1, keepdims=True)
            d = (sq_row + sq_col) - 2.0 * mm
            brow = brow_ref[pl.ds(r0, R), :]
            d_scr[:] = jnp.where(brow != bcol, jnp.float32(1e10), d)

            def pick(t, carry):
                acc, sel_prev = carry
                dd = jnp.where(col_iota == sel_prev, jnp.float32(jnp.inf),
                               d_scr[:])
                d_scr[:] = dd
                m = jnp.min(dd, axis=1, keepdims=True)
                sel = jnp.min(jnp.where(dd == m, col_iota, jnp.int32(2**30)),
                              axis=1, keepdims=True)
                acc = jnp.where(k_iota == t, sel, acc)
                return acc, sel

            acc, _ = lax.fori_loop(
                0, KNN, pick,
                (jnp.zeros((R, KNN), jnp.int32),
                 jnp.full((R, 1), -1, jnp.int32)))
            old = idx_ref[pl.ds(r0, R), :]
            keep = brow == b
            idx_ref[pl.ds(r0, R), :] = jnp.where(keep, acc + w0, old)
            return 0

        lax.fori_loop(0, seg_ref[4 + b], block, 0)


def _edge_max(x, a, xj_hbm, wb, d_in, slab, sems):
    def cp(j, buf):
        return pltpu.make_async_copy(xj_hbm.at[pl.ds(j * N, N)],
                                     slab.at[buf], sems.at[buf])

    cp(0, 0).start()

    def step(j, acc):
        buf = j % 2

        @pl.when(j + 1 < KNN)
        def _():
            cp(j + 1, 1 - buf).start()

        cp(j, buf).wait()
        xj = slab[buf][:, 0:d_in]
        msg = jnp.maximum(a + _dot(xj - x, wb), 0.0)
        return jnp.maximum(acc, msg)

    init = jnp.full((N, a.shape[1]), -jnp.inf, jnp.float32)
    return lax.fori_loop(0, KNN, step, init)


def _first_body(pos_ref, brow_ref, bcol_ref, w0_ref, w1_ref, b1_ref, wl1_ref,
                bl1_ref, idx_ref, x_ref, a_ref, x64_scr, xt_scr, sqc_scr,
                d_scr):
    x = jnp.maximum(_dot(pos_ref[:], w1_ref[:]) + b1_ref[:], 0.0)
    x_ref[:, 0:64] = x
    x_ref[:, 64:128] = jnp.zeros((N, 64), jnp.float32)
    x64_scr[:] = x
    xt_scr[:] = x.T
    a_ref[:] = _dot(x, wl1_ref[0:64, :]) + bl1_ref[:]
    _knn(x64_scr, xt_scr, sqc_scr, d_scr, brow_ref, bcol_ref, w0_ref,
         idx_ref, 512)


def _mid_body(x1_ref, a1_ref, xj_ref, brow_ref, bcol_ref, w0_ref, wl1_ref,
              wl2_ref, bl2_ref, idx_ref, x_ref, a_ref, xt_scr, sqc_scr,
              d_scr, slab, sems):
    x2 = _edge_max(x1_ref[:, 0:64], a1_ref[:], xj_ref, wl1_ref[64:128, :],
                   64, slab, sems)
    x_ref[:] = x2
    xt_scr[:] = x2.T
    a_ref[:] = _dot(x2, wl2_ref[0:128, :]) + bl2_ref[:]
    _knn(x_ref, xt_scr, sqc_scr, d_scr, brow_ref, bcol_ref, w0_ref,
         idx_ref, 512)


def _last_body(x2_ref, a2_ref, xj_ref, wl2_ref, w2_ref, b2_ref, o_ref,
               slab, sems):
    x3 = _edge_max(x2_ref[:], a2_ref[:], xj_ref, wl2_ref[128:256, :],
                   128, slab, sems)
    o_ref[:] = _dot(x3, w2_ref[:]) + b2_ref[:]


_VMEM_SPEC = pl.BlockSpec(memory_space=pltpu.MemorySpace.VMEM)
_SMEM_SPEC = pl.BlockSpec(memory_space=pltpu.MemorySpace.SMEM)
_ANY_SPEC = pl.BlockSpec(memory_space=pl.ANY)

_TC_FIRST = pl.pallas_call(
    _first_body,
    in_specs=[_VMEM_SPEC, _VMEM_SPEC, _VMEM_SPEC, _SMEM_SPEC,
              _VMEM_SPEC, _VMEM_SPEC, _VMEM_SPEC, _VMEM_SPEC],
    out_shape=(
        jax.ShapeDtypeStruct((N, KNN), jnp.int32),
        jax.ShapeDtypeStruct((N, 128), jnp.float32),
        jax.ShapeDtypeStruct((N, 128), jnp.float32),
    ),
    scratch_shapes=[
        pltpu.VMEM((N, 64), jnp.float32),
        pltpu.VMEM((64, N), jnp.float32),
        pltpu.VMEM((1, N), jnp.float32),
        pltpu.VMEM((512, WSEG), jnp.float32),
    ],
)

_TC_MID = pl.pallas_call(
    _mid_body,
    in_specs=[_VMEM_SPEC, _VMEM_SPEC, _ANY_SPEC, _VMEM_SPEC, _VMEM_SPEC,
              _SMEM_SPEC, _VMEM_SPEC, _VMEM_SPEC, _VMEM_SPEC],
    out_shape=(
        jax.ShapeDtypeStruct((N, KNN), jnp.int32),
        jax.ShapeDtypeStruct((N, 128), jnp.float32),
        jax.ShapeDtypeStruct((N, 256), jnp.float32),
    ),
    scratch_shapes=[
        pltpu.VMEM((128, N), jnp.float32),
        pltpu.VMEM((1, N), jnp.float32),
        pltpu.VMEM((512, WSEG), jnp.float32),
        pltpu.VMEM((2, N, 128), jnp.float32),
        pltpu.SemaphoreType.DMA((2,)),
    ],
)

_TC_LAST = pl.pallas_call(
    _last_body,
    in_specs=[_VMEM_SPEC, _VMEM_SPEC, _ANY_SPEC,
              _VMEM_SPEC, _VMEM_SPEC, _VMEM_SPEC],
    out_shape=jax.ShapeDtypeStruct((N, 128), jnp.float32),
    scratch_shapes=[
        pltpu.VMEM((2, N, 128), jnp.float32),
        pltpu.SemaphoreType.DMA((2,)),
    ],
)


def _make_gather(D):
    G = N * KNN
    GPW = G // _SC_WORKERS
    CH = 128
    NCHUNK = GPW // CH
    mesh = plsc.VectorSubcoreMesh(core_axis_name="c", subcore_axis_name="s",
                                  num_cores=_SC_CORES,
                                  num_subcores=_SC_SUBCORES)

    @functools.partial(
        pl.kernel, mesh=mesh,
        out_type=jax.ShapeDtypeStruct((G, D), jnp.float32),
        scratch_types=[
            pltpu.VMEM((CH,), jnp.int32),
            pltpu.VMEM((CH, D), jnp.float32),
            pltpu.SemaphoreType.DMA,
        ],
    )
    def gather(idx_hbm, table_hbm, out_hbm, idx_v, rows_v, sem):
        wid = lax.axis_index("s") * _SC_CORES + lax.axis_index("c")
        base = wid * GPW

        def chunk(ci, _):
            g0 = base + ci * CH
            pltpu.sync_copy(idx_hbm.at[pl.ds(g0, CH)], idx_v)
            pltpu.async_copy(table_hbm.at[idx_v], rows_v, sem).wait()
            pltpu.sync_copy(rows_v, out_hbm.at[pl.ds(g0, CH)])
            return 0

        lax.fori_loop(0, NCHUNK, chunk, 0)

    return gather


_GATHER128 = functools.cache(lambda: _make_gather(128))


def kernel(pos, batch, W1, b1, Wl1, bl1, Wl2, bl2, W2, b2):
    brow = batch.reshape(N, 1)
    bcol = batch.reshape(1, N)
    seg_start = jnp.searchsorted(batch, jnp.arange(4, dtype=batch.dtype))
    seg_end = jnp.concatenate([seg_start[1:],
                               jnp.array([N], seg_start.dtype)])
    w0 = jnp.minimum((seg_start // 128) * 128, N - WSEG).astype(jnp.int32)
    nb = ((seg_end.astype(jnp.int32) - w0) + 511) // 512
    w0 = jnp.concatenate([w0, nb]).astype(jnp.int32)
    idx1, x1, A1 = _TC_FIRST(pos, brow, bcol, w0, W1, b1.reshape(1, -1),
                             Wl1, bl1.reshape(1, -1))
    XJ1 = _GATHER128()(idx1.T.reshape(-1), x1)
    idx2, x2, A2 = _TC_MID(x1, A1, XJ1, brow, bcol, w0, Wl1, Wl2,
                           bl2.reshape(1, -1))
    XJ2 = _GATHER128()(idx2.T.reshape(-1), x2)
    return _TC_LAST(x2, A2, XJ2, Wl2, W2, b2.reshape(1, -1))
